# baseline (speedup 1.0000x reference)
import functools
import math

import jax
import jax.numpy as jnp
from jax import lax
from jax.experimental import pallas as pl
from jax.experimental.pallas import tpu as pltpu
from jax.experimental.pallas import tpu_sc as plsc

D_MODEL = 64
SCALE = math.sqrt(D_MODEL)

NC = 2
NS = 16
NW = NC * NS
BLK = 200
GROUPS = tuple(list(range(0, BLK - 16, 16)) + [BLK - 16])


def _bcast(vec, lane):
    dn = lax.GatherDimensionNumbers(
        offset_dims=(), collapsed_slice_dims=(0,), start_index_map=(0,)
    )
    idx = jnp.full((16, 1), lane, jnp.int32)
    return lax.gather(
        vec, idx, dn, slice_sizes=(1,),
        mode=lax.GatherScatterMode.PROMISE_IN_BOUNDS,
    )


@functools.partial(jax.jit, static_argnums=(2, 3))
def _embed(xf, tp, a_dim, b_dim):
    B = xf.shape[0]
    b_per_w = B // NW
    blocks_per_w = b_per_w // BLK

    mesh = plsc.VectorSubcoreMesh(core_axis_name="c", subcore_axis_name="s")

    @functools.partial(
        pl.kernel,
        mesh=mesh,
        out_type=jax.ShapeDtypeStruct((a_dim, b_dim, D_MODEL), jnp.float32),
        scratch_types=[
            pltpu.VMEM((b_per_w,), jnp.int32),
            pltpu.VMEM((2, 256), jnp.int32),
            pltpu.VMEM((2, BLK, 128), jnp.float32),
            pltpu.VMEM((2, BLK, D_MODEL), jnp.float32),
            pltpu.SemaphoreType.DMA,
            pltpu.SemaphoreType.DMA,
            pltpu.SemaphoreType.DMA,
            pltpu.SemaphoreType.DMA,
        ],
        compiler_params=pltpu.CompilerParams(
            use_tc_tiling_on_sc=True, needs_layout_passes=False
        ),
    )
    def body(xf_hbm, tp_hbm, out_hbm, iv, pv, rowsg, out64,
             gsem0, gsem1, wsem0, wsem1):
        wid = lax.axis_index("s") * NC + lax.axis_index("c")
        row_base = wid * b_per_w
        blk_base = wid * blocks_per_w
        gsems = (gsem0, gsem1)
        wsems = (wsem0, wsem1)

        pltpu.sync_copy(xf_hbm.at[pl.ds(row_base, b_per_w)], iv)

        def fire_gathers(a, s):
            base = a * BLK

            def pv_body(g, c2):
                gs = g * 16
                pv[s, pl.ds(gs, 16)] = iv[pl.ds(base + gs, 16)] >> 1
                return c2

            lax.fori_loop(0, BLK // 16, pv_body, 0)
            pv[s, pl.ds(BLK - 16, 16)] = iv[pl.ds(base + BLK - 16, 16)] >> 1
            pltpu.async_copy(
                tp_hbm.at[pv.at[s, pl.ds(0, 128)]],
                rowsg.at[s, pl.ds(0, 128)],
                gsems[s],
            )
            pltpu.async_copy(
                tp_hbm.at[pv.at[s, pl.ds(128, BLK - 128)]],
                rowsg.at[s, pl.ds(128, BLK - 128)],
                gsems[s],
            )

        def wait_gathers(s):
            pltpu.make_async_copy(
                tp_hbm.at[pl.ds(0, 128)], rowsg.at[s, pl.ds(0, 128)], gsems[s]
            ).wait()
            pltpu.make_async_copy(
                tp_hbm.at[pl.ds(0, BLK - 128)],
                rowsg.at[s, pl.ds(128, BLK - 128)],
                gsems[s],
            ).wait()

        def wait_write(s):
            pltpu.make_async_copy(
                out64.at[s], out_hbm.at[blk_base], wsems[s]
            ).wait()

        fire_gathers(0, 0)

        def super_body(t, carry):
            for par in (0, 1):
                a = 2 * t + par

                @pl.when(a + 1 < blocks_per_w)
                def _():
                    fire_gathers(a + 1, 1 - par)

                wait_gathers(par)

                @pl.when(a >= 2)
                def _():
                    wait_write(par)

                base = a * BLK

                def hsel(gs):
                    h16 = iv[pl.ds(base + gs, 16)] & 1
                    for j in range(16):
                        hb = _bcast(h16, j)
                        r = gs + j
                        for k in range(D_MODEL // 16):
                            alo = rowsg[par, r, pl.ds(k * 16, 16)]
                            ahi = rowsg[par, r, pl.ds(64 + k * 16, 16)]
                            out64[par, r, pl.ds(k * 16, 16)] = jnp.where(
                                hb > 0, ahi, alo
                            )

                def group_body(g, c2):
                    hsel(g * 16)
                    return c2

                lax.fori_loop(0, BLK // 16, group_body, 0)
                hsel(BLK - 16)

                pltpu.async_copy(
                    out64.at[par], out_hbm.at[blk_base + a], wsems[par]
                )
            return carry

        lax.fori_loop(0, blocks_per_w // 2, super_body, 0)
        wait_write(0)
        wait_write(1)

    return body(xf, tp)


def kernel(x, table):
    xf = x.reshape(-1).astype(jnp.int32)
    tp = (table * SCALE).reshape(table.shape[0] // 2, 2 * table.shape[1])
    return _embed(xf, tp, x.shape[0], x.shape[1])


# speedup vs baseline: 1.2560x; 1.2560x over previous
import functools
import math

import jax
import jax.numpy as jnp
from jax import lax
from jax.experimental import pallas as pl
from jax.experimental.pallas import tpu as pltpu
from jax.experimental.pallas import tpu_sc as plsc

D_MODEL = 64
SCALE = math.sqrt(D_MODEL)

NC = 2
NS = 16
NW = NC * NS
BLK = 200


def _bcast(vec, lane):
    dn = lax.GatherDimensionNumbers(
        offset_dims=(), collapsed_slice_dims=(0,), start_index_map=(0,)
    )
    idx = jnp.full((16, 1), lane, jnp.int32)
    return lax.gather(
        vec, idx, dn, slice_sizes=(1,),
        mode=lax.GatherScatterMode.PROMISE_IN_BOUNDS,
    )


@functools.partial(jax.jit, static_argnums=(2, 3))
def _embed(xf, tp, a_dim, b_dim):
    B = xf.shape[0]
    b_per_w = B // NW
    blocks_per_w = b_per_w // BLK

    mesh = plsc.VectorSubcoreMesh(core_axis_name="c", subcore_axis_name="s")

    @functools.partial(
        pl.kernel,
        mesh=mesh,
        out_type=jax.ShapeDtypeStruct((a_dim * b_dim, D_MODEL), jnp.float32),
        scratch_types=[
            pltpu.VMEM((b_per_w,), jnp.int32),
            pltpu.VMEM((2, 256), jnp.int32),
            pltpu.VMEM((2, BLK, 128), jnp.float32),
            pltpu.VMEM((2, BLK, D_MODEL), jnp.float32),
            pltpu.SemaphoreType.DMA,
            pltpu.SemaphoreType.DMA,
            pltpu.SemaphoreType.DMA,
            pltpu.SemaphoreType.DMA,
        ],
        compiler_params=pltpu.CompilerParams(
            use_tc_tiling_on_sc=True, needs_layout_passes=False
        ),
    )
    def body(xf_hbm, tp_hbm, out_hbm, iv, pv, rowsg, out64,
             gsem0, gsem1, wsem0, wsem1):
        wid = lax.axis_index("s") * NC + lax.axis_index("c")
        row_base = wid * b_per_w
        blk_base = wid * blocks_per_w
        gsems = (gsem0, gsem1)
        wsems = (wsem0, wsem1)

        pltpu.sync_copy(xf_hbm.at[pl.ds(row_base, b_per_w)], iv)

        def fire_gathers(a, s):
            base = a * BLK

            def pv_body(g, c2):
                gs = g * 16
                pv[s, pl.ds(gs, 16)] = iv[pl.ds(base + gs, 16)] >> 1
                return c2

            lax.fori_loop(0, BLK // 16, pv_body, 0)
            pv[s, pl.ds(BLK - 16, 16)] = iv[pl.ds(base + BLK - 16, 16)] >> 1
            pltpu.async_copy(
                tp_hbm.at[pv.at[s, pl.ds(0, 128)]],
                rowsg.at[s, pl.ds(0, 128)],
                gsems[s],
            )
            pltpu.async_copy(
                tp_hbm.at[pv.at[s, pl.ds(128, BLK - 128)]],
                rowsg.at[s, pl.ds(128, BLK - 128)],
                gsems[s],
            )

        def wait_gathers(s):
            pltpu.make_async_copy(
                tp_hbm.at[pl.ds(0, 128)], rowsg.at[s, pl.ds(0, 128)], gsems[s]
            ).wait()
            pltpu.make_async_copy(
                tp_hbm.at[pl.ds(0, BLK - 128)],
                rowsg.at[s, pl.ds(128, BLK - 128)],
                gsems[s],
            ).wait()

        def wait_write(s):
            pltpu.make_async_copy(
                out64.at[s], out_hbm.at[pl.ds(row_base, BLK)], wsems[s]
            ).wait()

        fire_gathers(0, 0)

        def super_body(t, carry):
            for par in (0, 1):
                a = 2 * t + par

                @pl.when(a + 1 < blocks_per_w)
                def _():
                    fire_gathers(a + 1, 1 - par)

                wait_gathers(par)

                @pl.when(a >= 2)
                def _():
                    wait_write(par)

                base = a * BLK

                def hsel(gs):
                    h16 = iv[pl.ds(base + gs, 16)] & 1
                    for j in range(16):
                        hb = _bcast(h16, j)
                        r = gs + j
                        for k in range(D_MODEL // 16):
                            alo = rowsg[par, r, pl.ds(k * 16, 16)]
                            ahi = rowsg[par, r, pl.ds(64 + k * 16, 16)]
                            out64[par, r, pl.ds(k * 16, 16)] = (
                                jnp.where(hb > 0, ahi, alo) * SCALE
                            )

                def group_body(g, c2):
                    hsel(g * 16)
                    return c2

                lax.fori_loop(0, BLK // 16, group_body, 0)
                hsel(BLK - 16)

                pltpu.async_copy(
                    out64.at[par],
                    out_hbm.at[pl.ds(row_base + a * BLK, BLK)],
                    wsems[par],
                )
            return carry

        lax.fori_loop(0, blocks_per_w // 2, super_body, 0)
        wait_write(0)
        wait_write(1)

    return body(xf, tp)


def kernel(x, table):
    xf = x.reshape(-1).astype(jnp.int32)
    tp = table.reshape(table.shape[0] // 2, 2 * table.shape[1])
    out = _embed(xf, tp, x.shape[0], x.shape[1])
    return out.reshape(x.shape + (table.shape[1],))
